# baseline (speedup 1.0000x reference)
import functools

import jax
import jax.numpy as jnp
from jax import lax
from jax.experimental import pallas as pl
from jax.experimental.pallas import tpu as pltpu
from jax.experimental.pallas import tpu_sc as plsc

NF = 128
NF2 = 256
DK = 64
T = 16

NP = 10240
NTILES = 32
EPT = 5120
EP = EPT * NTILES
CHUNK = 128
NCH = EPT // CHUNK

_HI = lax.Precision.HIGHEST


def _lstm_body(x_ref, web_ref, beb_ref, wih_ref, whh_ref, bl_ref, out_ref):
    B = out_ref.shape[0]
    web = web_ref[...]
    wih = wih_ref[...]
    whh = whh_ref[...]
    beb = beb_ref[...]
    bl = bl_ref[...]
    h = jnp.zeros((B, NF), jnp.float32)
    c = jnp.zeros((B, NF), jnp.float32)
    for t in range(T):
        xt = x_ref[t]
        ht = jnp.maximum(jnp.dot(xt, web, precision=_HI) + beb, 0.0)
        g = (jnp.dot(ht, wih, precision=_HI)
             + jnp.dot(h, whh, precision=_HI) + bl)
        i = jax.nn.sigmoid(g[:, :NF])
        f = jax.nn.sigmoid(g[:, NF:2 * NF])
        gg = jnp.tanh(g[:, 2 * NF:3 * NF])
        o = jax.nn.sigmoid(g[:, 3 * NF:])
        c = f * c + i * gg
        h = o * jnp.tanh(c)
    out_ref[...] = h


def _lstm_last(x, W_eb, b_eb, W_ih, W_hh, b_lstm):
    npat = x.shape[0]
    B = 1000
    xT = jnp.transpose(x, (1, 0, 2))
    grid = (npat // B,)
    return pl.pallas_call(
        _lstm_body,
        grid=grid,
        in_specs=[
            pl.BlockSpec((T, B, NF), lambda g: (0, g, 0)),
            pl.BlockSpec((NF, NF), lambda g: (0, 0)),
            pl.BlockSpec((1, NF), lambda g: (0, 0)),
            pl.BlockSpec((NF, 4 * NF), lambda g: (0, 0)),
            pl.BlockSpec((NF, 4 * NF), lambda g: (0, 0)),
            pl.BlockSpec((1, 4 * NF), lambda g: (0, 0)),
        ],
        out_specs=pl.BlockSpec((B, NF), lambda g: (g, 0)),
        out_shape=jax.ShapeDtypeStruct((npat, NF), jnp.float32),
    )(xT, W_eb, b_eb.reshape(1, NF), W_ih, W_hh, b_lstm.reshape(1, 4 * NF))


def _zmm_body(x_ref, w_ref, a2_ref, z0_ref, z1_ref, z2_ref, z3_ref, ea_ref):
    z = jnp.dot(x_ref[...], w_ref[...], precision=_HI)
    z0_ref[...] = z[:, 0:64]
    z1_ref[...] = z[:, 64:128]
    z2_ref[...] = z[:, 128:192]
    z3_ref[...] = z[:, 192:256]
    ea_ref[...] = jnp.dot(z, a2_ref[...], precision=_HI)


def _gat_dense(x_pad, W, a_l, a_r):
    B = 512
    a2 = jnp.zeros((NF2, 128), jnp.float32)
    a2 = a2.at[:, 0].set(a_l).at[:, 1].set(a_r)
    zq = pl.pallas_call(
        _zmm_body,
        grid=(NP // B,),
        in_specs=[
            pl.BlockSpec((B, NF2), lambda g: (g, 0)),
            pl.BlockSpec((NF2, NF2), lambda g: (0, 0)),
            pl.BlockSpec((NF2, 128), lambda g: (0, 0)),
        ],
        out_specs=[pl.BlockSpec((B, 64), lambda g: (g, 0))] * 4
        + [pl.BlockSpec((B, 128), lambda g: (g, 0))],
        out_shape=[jax.ShapeDtypeStruct((NP, 64), jnp.float32)] * 4
        + [jax.ShapeDtypeStruct((NP, 128), jnp.float32)],
    )(x_pad, W, a2)
    return zq[0], zq[1], zq[2], zq[3], zq[4][:, 0], zq[4][:, 1]


def _edge_body(z0, z1, z2, z3, el_h, er_h, src_h, dst_h,
               oq0, oq1, oq2, oq3, denp,
               src_v, dst_v, el_v, er_v, ex_v, den_v, sidx_v, didx_v,
               rows_v, zero_v, acc_sh, sem):
    cid = lax.axis_index("c")
    sid = lax.axis_index("s")
    tid = sid * 2 + cid
    base = tid * EPT
    pltpu.sync_copy(src_h.at[pl.ds(base, EPT)], src_v)
    pltpu.sync_copy(dst_h.at[pl.ds(base, EPT)], dst_v)
    pltpu.sync_copy(el_h, el_v)
    pltpu.sync_copy(er_h, er_v)

    zeros16 = jnp.zeros((16,), jnp.float32)

    def _zden(i, carry):
        den_v[pl.ds(i * 16, 16)] = zeros16
        return carry

    lax.fori_loop(0, NP // 16, _zden, 0)

    def _zrow(r, carry):
        for j in range(4):
            zero_v[r, pl.ds(j * 16, 16)] = zeros16
        return carry

    lax.fori_loop(0, CHUNK, _zrow, 0)

    def _pa(i, carry):
        s = src_v[pl.ds(i * 16, 16)]
        d = dst_v[pl.ds(i * 16, 16)]
        e = plsc.load_gather(el_v, [s]) + plsc.load_gather(er_v, [d])
        e = jnp.where(e > 0, e, 0.2 * e)
        ex = jnp.exp(e)
        ex_v[pl.ds(i * 16, 16)] = ex
        plsc.addupdate_scatter(den_v, [d], ex)
        return carry

    lax.fori_loop(0, EPT // 16, _pa, 0)
    pltpu.sync_copy(den_v, denp.at[tid])

    for h in range(4):
        zh = (z0, z1, z2, z3)[h]
        plsc.subcore_barrier()
        for j in range(5):
            pltpu.sync_copy(zero_v, acc_sh.at[pl.ds((sid * 5 + j) * CHUNK, CHUNK)])
        plsc.subcore_barrier()

        def _pb(c, carry):
            for j in range(CHUNK // 16):
                sidx_v[pl.ds(j * 16, 16)] = src_v[pl.ds(c * CHUNK + j * 16, 16)]
                didx_v[pl.ds(j * 16, 16)] = dst_v[pl.ds(c * CHUNK + j * 16, 16)]
            pltpu.async_copy(zh.at[sidx_v], rows_v, sem).wait()

            def _row(r, carry2):
                sp = plsc.load_gather(ex_v, [lax.broadcast(c * CHUNK + r, (16,))])
                for j in range(4):
                    rows_v[r, pl.ds(j * 16, 16)] = rows_v[r, pl.ds(j * 16, 16)] * sp
                return carry2

            lax.fori_loop(0, CHUNK, _row, 0)
            pltpu.sync_copy(rows_v, acc_sh.at[didx_v], add=True)
            return carry

        lax.fori_loop(0, NCH, _pb, 0)
        plsc.subcore_barrier()
        oq = (oq0, oq1, oq2, oq3)[h]
        pltpu.sync_copy(
            acc_sh.at[pl.ds(sid * (NP // 16), NP // 16)],
            oq.at[cid, pl.ds(sid * (NP // 16), NP // 16)])
    plsc.subcore_barrier()


@functools.cache
def _edge_sc_kernel():
  return pl.kernel(
    _edge_body,
    out_type=(
        jax.ShapeDtypeStruct((2, NP, 64), jnp.float32),
        jax.ShapeDtypeStruct((2, NP, 64), jnp.float32),
        jax.ShapeDtypeStruct((2, NP, 64), jnp.float32),
        jax.ShapeDtypeStruct((2, NP, 64), jnp.float32),
        jax.ShapeDtypeStruct((NTILES, NP), jnp.float32),
    ),
    mesh=plsc.VectorSubcoreMesh(core_axis_name="c", subcore_axis_name="s"),
    compiler_params=pltpu.CompilerParams(needs_layout_passes=False,
                                         use_tc_tiling_on_sc=False),
    scratch_types=[
        pltpu.VMEM((EPT,), jnp.int32),
        pltpu.VMEM((EPT,), jnp.int32),
        pltpu.VMEM((NP,), jnp.float32),
        pltpu.VMEM((NP,), jnp.float32),
        pltpu.VMEM((EPT,), jnp.float32),
        pltpu.VMEM((NP,), jnp.float32),
        pltpu.VMEM((CHUNK,), jnp.int32),
        pltpu.VMEM((CHUNK,), jnp.int32),
        pltpu.VMEM((CHUNK, 64), jnp.float32),
        pltpu.VMEM((CHUNK, 64), jnp.float32),
        pltpu.VMEM_SHARED((NP, 64), jnp.float32),
        pltpu.SemaphoreType.DMA,
    ],
  )


def _lrelu(x, slope=0.01):
    return jnp.where(x > 0, x, slope * x)


def _gat_pallas(x_pad, src_p, dst_p, W, a_l, a_r, b):
    z0, z1, z2, z3, el, er = _gat_dense(x_pad, W, a_l, a_r)
    oq0, oq1, oq2, oq3, denp = _edge_sc_kernel()(
        z0, z1, z2, z3, el, er, src_p, dst_p)
    den = jnp.sum(denp, axis=0)
    den = jnp.where(den > 0, den, 1.0)
    outs = jnp.concatenate(
        [oq0[0] + oq0[1], oq1[0] + oq1[1], oq2[0] + oq2[1], oq3[0] + oq3[1]],
        axis=1)
    out = outs / den[:, None] + b
    return out


def _pad_edges(src, dst):
    pad = EP - src.shape[0]
    padv = jnp.full((pad,), NP - 1, jnp.int32)
    return (jnp.concatenate([src.astype(jnp.int32), padv]),
            jnp.concatenate([dst.astype(jnp.int32), padv]))


def kernel(input_tensor, static_tensor, W_eb, b_eb, W_ih, W_hh, b_lstm, Wg, al,
           ar, bg, Wq, bq, Wk, bk, Wo, bo, ei0, ei1, ei2, ei3, ei4, ei5, w):
    npat = input_tensor.shape[0]

    aft = _lstm_last(input_tensor, W_eb, b_eb, W_ih, W_hh, b_lstm)

    hdet = aft
    sq = jnp.sum(hdet * hdet, axis=1)
    d2 = sq[:, None] + sq[None, :] - 2.0 * (hdet @ hdet.T)
    _, idx = jax.lax.top_k(-d2, 16)
    src_dy = jnp.repeat(jnp.arange(npat, dtype=jnp.int32), 16)
    dst_dy = idx.reshape(-1).astype(jnp.int32)

    x = jnp.concatenate([aft, static_tensor], axis=1)
    x_pad = jnp.concatenate(
        [x, jnp.zeros((NP - npat, NF2), dtype=x.dtype)], axis=0)

    e_dy = _pad_edges(src_dy, dst_dy)
    e0 = _pad_edges(ei0[0], ei0[1])
    e1 = _pad_edges(ei1[0], ei1[1])
    e2 = _pad_edges(ei2[0], ei2[1])
    e3 = _pad_edges(ei3[0], ei3[1])
    e4 = _pad_edges(ei4[0], ei4[1])
    e5 = _pad_edges(ei5[0], ei5[1])

    aft_dy = _lrelu(_gat_pallas(x_pad, *e_dy, Wg[0], al[0], ar[0], bg[0]))
    g1 = _lrelu(_gat_pallas(x_pad, *e0, Wg[1], al[1], ar[1], bg[1]))
    g2 = _lrelu(_gat_pallas(x_pad, *e1, Wg[2], al[2], ar[2], bg[2]))
    g3 = _lrelu(_gat_pallas(x_pad, *e2, Wg[3], al[3], ar[3], bg[3]))
    g3 = _lrelu(_gat_pallas(g3, *e3, Wg[4], al[4], ar[4], bg[4]))
    g4 = _lrelu(_gat_pallas(x_pad, *e4, Wg[5], al[5], ar[5], bg[5]))
    g4 = _lrelu(_gat_pallas(g4, *e5, Wg[6], al[6], ar[6], bg[6]))

    X = jnp.stack([g1[:npat], g2[:npat], g3[:npat], g4[:npat],
                   aft_dy[:npat]], axis=1)
    hq = x[:, None, :]
    Q = hq @ Wq + bq
    K = X @ Wk + bk
    A = jax.nn.softmax(
        jnp.matmul(Q, jnp.swapaxes(K, -1, -2)) / jnp.sqrt(jnp.float32(DK)),
        axis=2)
    merged = jnp.matmul(A, X).reshape(npat, NF2)
    bft = jnp.concatenate([x, merged], axis=1)
    out = bft @ Wo + bo
    return jax.nn.log_softmax(out, axis=1)


# speedup vs baseline: 2.0997x; 2.0997x over previous
import functools

import jax
import jax.numpy as jnp
from jax import lax
from jax.experimental import pallas as pl
from jax.experimental.pallas import tpu as pltpu
from jax.experimental.pallas import tpu_sc as plsc

NF = 128
NF2 = 256
DK = 64
T = 16

NP = 10240
NTILES = 32
EPT = 5120
EP = EPT * NTILES
CHUNK = 128
NCH = EPT // CHUNK

_HI = lax.Precision.HIGHEST


def _lstm_body(x_ref, web_ref, beb_ref, wih_ref, whh_ref, bl_ref, out_ref):
    B = out_ref.shape[0]
    web = web_ref[...]
    wih = wih_ref[...]
    whh = whh_ref[...]
    beb = beb_ref[...]
    bl = bl_ref[...]
    h = jnp.zeros((B, NF), jnp.float32)
    c = jnp.zeros((B, NF), jnp.float32)
    for t in range(T):
        xt = x_ref[t]
        ht = jnp.maximum(jnp.dot(xt, web, precision=_HI) + beb, 0.0)
        g = (jnp.dot(ht, wih, precision=_HI)
             + jnp.dot(h, whh, precision=_HI) + bl)
        i = jax.nn.sigmoid(g[:, :NF])
        f = jax.nn.sigmoid(g[:, NF:2 * NF])
        gg = jnp.tanh(g[:, 2 * NF:3 * NF])
        o = jax.nn.sigmoid(g[:, 3 * NF:])
        c = f * c + i * gg
        h = o * jnp.tanh(c)
    out_ref[...] = h


def _lstm_last(x, W_eb, b_eb, W_ih, W_hh, b_lstm):
    npat = x.shape[0]
    B = 1000
    xT = jnp.transpose(x, (1, 0, 2))
    grid = (npat // B,)
    return pl.pallas_call(
        _lstm_body,
        grid=grid,
        in_specs=[
            pl.BlockSpec((T, B, NF), lambda g: (0, g, 0)),
            pl.BlockSpec((NF, NF), lambda g: (0, 0)),
            pl.BlockSpec((1, NF), lambda g: (0, 0)),
            pl.BlockSpec((NF, 4 * NF), lambda g: (0, 0)),
            pl.BlockSpec((NF, 4 * NF), lambda g: (0, 0)),
            pl.BlockSpec((1, 4 * NF), lambda g: (0, 0)),
        ],
        out_specs=pl.BlockSpec((B, NF), lambda g: (g, 0)),
        out_shape=jax.ShapeDtypeStruct((npat, NF), jnp.float32),
    )(xT, W_eb, b_eb.reshape(1, NF), W_ih, W_hh, b_lstm.reshape(1, 4 * NF))


def _zmm_body(x_ref, w_ref, a2_ref, z0_ref, z1_ref, z2_ref, z3_ref, ea_ref):
    z = jnp.dot(x_ref[...], w_ref[...], precision=_HI)
    z0_ref[...] = z[:, 0:64]
    z1_ref[...] = z[:, 64:128]
    z2_ref[...] = z[:, 128:192]
    z3_ref[...] = z[:, 192:256]
    ea_ref[...] = jnp.dot(z, a2_ref[...], precision=_HI)


def _gat_dense(x_pad, W, a_l, a_r):
    B = 512
    a2 = jnp.zeros((NF2, 128), jnp.float32)
    a2 = a2.at[:, 0].set(a_l).at[:, 1].set(a_r)
    zq = pl.pallas_call(
        _zmm_body,
        grid=(NP // B,),
        in_specs=[
            pl.BlockSpec((B, NF2), lambda g: (g, 0)),
            pl.BlockSpec((NF2, NF2), lambda g: (0, 0)),
            pl.BlockSpec((NF2, 128), lambda g: (0, 0)),
        ],
        out_specs=[pl.BlockSpec((B, 64), lambda g: (g, 0))] * 4
        + [pl.BlockSpec((B, 128), lambda g: (g, 0))],
        out_shape=[jax.ShapeDtypeStruct((NP, 64), jnp.float32)] * 4
        + [jax.ShapeDtypeStruct((NP, 128), jnp.float32)],
    )(x_pad, W, a2)
    return zq[0], zq[1], zq[2], zq[3], zq[4][:, 0], zq[4][:, 1]


def _edge_body(z0, z1, z2, z3, el_h, er_h, src_h, dst_h,
               oq0, oq1, oq2, oq3, denp,
               src_v, dst_v, el_v, er_v, ex_v, den_v, sidx_v, didx_v,
               rows_v, zero_v, acc_sh, sem):
    cid = lax.axis_index("c")
    sid = lax.axis_index("s")
    tid = sid * 2 + cid
    base = tid * EPT
    pltpu.sync_copy(src_h.at[pl.ds(base, EPT)], src_v)
    pltpu.sync_copy(dst_h.at[pl.ds(base, EPT)], dst_v)
    pltpu.sync_copy(el_h, el_v)
    pltpu.sync_copy(er_h, er_v)

    zeros16 = jnp.zeros((16,), jnp.float32)

    def _zden(i, carry):
        den_v[pl.ds(i * 16, 16)] = zeros16
        return carry

    lax.fori_loop(0, NP // 16, _zden, 0)

    def _zrow(r, carry):
        for j in range(4):
            zero_v[r, pl.ds(j * 16, 16)] = zeros16
        return carry

    lax.fori_loop(0, CHUNK, _zrow, 0)

    def _pa(i, carry):
        s = src_v[pl.ds(i * 16, 16)]
        d = dst_v[pl.ds(i * 16, 16)]
        e = plsc.load_gather(el_v, [s]) + plsc.load_gather(er_v, [d])
        e = jnp.where(e > 0, e, 0.2 * e)
        ex = jnp.exp(e)
        ex_v[pl.ds(i * 16, 16)] = ex
        plsc.addupdate_scatter(den_v, [d], ex)
        return carry

    lax.fori_loop(0, EPT // 16, _pa, 0)
    pltpu.sync_copy(den_v, denp.at[tid])

    for h in range(4):
        zh = (z0, z1, z2, z3)[h]
        plsc.subcore_barrier()
        for j in range(5):
            pltpu.sync_copy(zero_v, acc_sh.at[pl.ds((sid * 5 + j) * CHUNK, CHUNK)])
        plsc.subcore_barrier()

        def _pb(c, carry):
            for j in range(CHUNK // 16):
                sidx_v[pl.ds(j * 16, 16)] = src_v[pl.ds(c * CHUNK + j * 16, 16)]
                didx_v[pl.ds(j * 16, 16)] = dst_v[pl.ds(c * CHUNK + j * 16, 16)]
            pltpu.async_copy(zh.at[sidx_v], rows_v, sem).wait()

            def _row(r, carry2):
                sp = plsc.load_gather(ex_v, [lax.broadcast(c * CHUNK + r, (16,))])
                for j in range(4):
                    rows_v[r, pl.ds(j * 16, 16)] = rows_v[r, pl.ds(j * 16, 16)] * sp
                return carry2

            lax.fori_loop(0, CHUNK, _row, 0)
            pltpu.sync_copy(rows_v, acc_sh.at[didx_v], add=True)
            return carry

        lax.fori_loop(0, NCH, _pb, 0)
        plsc.subcore_barrier()
        oq = (oq0, oq1, oq2, oq3)[h]
        pltpu.sync_copy(
            acc_sh.at[pl.ds(sid * (NP // 16), NP // 16)],
            oq.at[cid, pl.ds(sid * (NP // 16), NP // 16)])
    plsc.subcore_barrier()


@functools.cache
def _edge_sc_kernel():
  return pl.kernel(
    _edge_body,
    out_type=(
        jax.ShapeDtypeStruct((2, NP, 64), jnp.float32),
        jax.ShapeDtypeStruct((2, NP, 64), jnp.float32),
        jax.ShapeDtypeStruct((2, NP, 64), jnp.float32),
        jax.ShapeDtypeStruct((2, NP, 64), jnp.float32),
        jax.ShapeDtypeStruct((NTILES, NP), jnp.float32),
    ),
    mesh=plsc.VectorSubcoreMesh(core_axis_name="c", subcore_axis_name="s"),
    compiler_params=pltpu.CompilerParams(needs_layout_passes=False,
                                         use_tc_tiling_on_sc=False),
    scratch_types=[
        pltpu.VMEM((EPT,), jnp.int32),
        pltpu.VMEM((EPT,), jnp.int32),
        pltpu.VMEM((NP,), jnp.float32),
        pltpu.VMEM((NP,), jnp.float32),
        pltpu.VMEM((EPT,), jnp.float32),
        pltpu.VMEM((NP,), jnp.float32),
        pltpu.VMEM((CHUNK,), jnp.int32),
        pltpu.VMEM((CHUNK,), jnp.int32),
        pltpu.VMEM((CHUNK, 64), jnp.float32),
        pltpu.VMEM((CHUNK, 64), jnp.float32),
        pltpu.VMEM_SHARED((NP, 64), jnp.float32),
        pltpu.SemaphoreType.DMA,
    ],
  )


def _lrelu(x, slope=0.01):
    return jnp.where(x > 0, x, slope * x)


def _gat_pallas(x_pad, src_p, dst_p, W, a_l, a_r, b):
    z0, z1, z2, z3, el, er = _gat_dense(x_pad, W, a_l, a_r)
    oq0, oq1, oq2, oq3, denp = _edge_sc_kernel()(
        z0, z1, z2, z3, el, er, src_p, dst_p)
    den = jnp.sum(denp, axis=0)
    den = jnp.where(den > 0, den, 1.0)
    outs = jnp.concatenate(
        [oq0[0] + oq0[1], oq1[0] + oq1[1], oq2[0] + oq2[1], oq3[0] + oq3[1]],
        axis=1)
    out = outs / den[:, None] + b
    return out


def _pad_edges(src, dst):
    pad = EP - src.shape[0]
    padv = jnp.full((pad,), NP - 1, jnp.int32)
    return (jnp.concatenate([src.astype(jnp.int32), padv]),
            jnp.concatenate([dst.astype(jnp.int32), padv]))


def _tala128(mat, idx_i):
    W = mat.shape[1]
    acc = None
    for q in range(W // 128):
        sub = mat[:, q * 128:(q + 1) * 128]
        loc = jnp.clip(idx_i - q * 128, 0, 127)
        g = jnp.take_along_axis(sub, loc, axis=1)
        sel = (idx_i >= q * 128) & (idx_i < (q + 1) * 128)
        acc = g if acc is None else jnp.where(sel, g, acc)
    return acc


def _knn_body(hb_ref, ht_ref, out_ref, d_ref):
    hb = hb_ref[...]
    ht = ht_ref[...]
    B = hb.shape[0]
    NC = ht.shape[1]
    G = 640
    NJ = NC // G
    BIGF = jnp.float32(3.0e38)
    sqr = jnp.sum(hb * hb, axis=1, keepdims=True)
    sqc = jnp.sum(ht * ht, axis=0, keepdims=True)
    colid = lax.broadcasted_iota(jnp.int32, (1, NC), 1).astype(jnp.float32)
    sqc = sqc + jnp.where(colid >= 10000.0, BIGF, 0.0)
    M = None
    for j in range(NJ):
        dj = ((sqr + sqc[:, j * G:(j + 1) * G])
              - 2.0 * jnp.dot(hb, ht[:, j * G:(j + 1) * G], precision=_HI))
        d_ref[:, j * G:(j + 1) * G] = dj
        M = dj if M is None else jnp.minimum(M, dj)
    iog = lax.broadcasted_iota(jnp.int32, (B, G), 1).astype(jnp.float32)
    gs = []
    for k in range(16):
        m = jnp.min(M, axis=1, keepdims=True)
        g = jnp.min(jnp.where(M == m, iog, BIGF), axis=1, keepdims=True)
        gs.append(g)
        M = jnp.where(iog == g, BIGF, M)
    Gm = jnp.concatenate(gs, axis=1)
    Gi = Gm.astype(jnp.int32)
    cs, cols = [], []
    for j in range(NJ):
        dj = d_ref[:, j * G:(j + 1) * G]
        cs.append(_tala128(dj, Gi))
        cols.append(Gm + jnp.float32(j * G))
    C = jnp.concatenate(cs, axis=1)
    COL = jnp.concatenate(cols, axis=1)
    ioc = lax.broadcasted_iota(jnp.int32, (B, 16 * NJ), 1).astype(jnp.float32)
    outs = []
    for k in range(16):
        m = jnp.min(C, axis=1, keepdims=True)
        p = jnp.min(jnp.where(C == m, ioc, BIGF), axis=1, keepdims=True)
        outs.append(_tala128(COL, p.astype(jnp.int32)))
        C = jnp.where(ioc == p, BIGF, C)
    idxf = jnp.concatenate(outs, axis=1)
    out_ref[...] = jnp.concatenate(
        [idxf.astype(jnp.int32), jnp.zeros((B, 112), jnp.int32)], axis=1)


def _knn_topk(aft):
    npat = aft.shape[0]
    hp = jnp.concatenate(
        [aft, jnp.zeros((NP - npat, NF), jnp.float32)], axis=0)
    ht = jnp.transpose(hp)
    B = 400
    out = pl.pallas_call(
        _knn_body,
        grid=(npat // B,),
        in_specs=[pl.BlockSpec((B, NF), lambda g: (g, 0)),
                  pl.BlockSpec((NF, NP), lambda g: (0, 0))],
        out_specs=pl.BlockSpec((B, 128), lambda g: (g, 0)),
        out_shape=jax.ShapeDtypeStruct((npat, 128), jnp.int32),
        scratch_shapes=[pltpu.VMEM((B, NP), jnp.float32)],
    )(hp, ht)
    return out[:, :16]


def kernel(input_tensor, static_tensor, W_eb, b_eb, W_ih, W_hh, b_lstm, Wg, al,
           ar, bg, Wq, bq, Wk, bk, Wo, bo, ei0, ei1, ei2, ei3, ei4, ei5, w):
    npat = input_tensor.shape[0]

    aft = _lstm_last(input_tensor, W_eb, b_eb, W_ih, W_hh, b_lstm)

    idx = _knn_topk(aft)
    src_dy = jnp.repeat(jnp.arange(npat, dtype=jnp.int32), 16)
    dst_dy = idx.reshape(-1)

    x = jnp.concatenate([aft, static_tensor], axis=1)
    x_pad = jnp.concatenate(
        [x, jnp.zeros((NP - npat, NF2), dtype=x.dtype)], axis=0)

    e_dy = _pad_edges(src_dy, dst_dy)
    e0 = _pad_edges(ei0[0], ei0[1])
    e1 = _pad_edges(ei1[0], ei1[1])
    e2 = _pad_edges(ei2[0], ei2[1])
    e3 = _pad_edges(ei3[0], ei3[1])
    e4 = _pad_edges(ei4[0], ei4[1])
    e5 = _pad_edges(ei5[0], ei5[1])

    aft_dy = _lrelu(_gat_pallas(x_pad, *e_dy, Wg[0], al[0], ar[0], bg[0]))
    g1 = _lrelu(_gat_pallas(x_pad, *e0, Wg[1], al[1], ar[1], bg[1]))
    g2 = _lrelu(_gat_pallas(x_pad, *e1, Wg[2], al[2], ar[2], bg[2]))
    g3 = _lrelu(_gat_pallas(x_pad, *e2, Wg[3], al[3], ar[3], bg[3]))
    g3 = _lrelu(_gat_pallas(g3, *e3, Wg[4], al[4], ar[4], bg[4]))
    g4 = _lrelu(_gat_pallas(x_pad, *e4, Wg[5], al[5], ar[5], bg[5]))
    g4 = _lrelu(_gat_pallas(g4, *e5, Wg[6], al[6], ar[6], bg[6]))

    X = jnp.stack([g1[:npat], g2[:npat], g3[:npat], g4[:npat],
                   aft_dy[:npat]], axis=1)
    hq = x[:, None, :]
    Q = hq @ Wq + bq
    K = X @ Wk + bk
    A = jax.nn.softmax(
        jnp.matmul(Q, jnp.swapaxes(K, -1, -2)) / jnp.sqrt(jnp.float32(DK)),
        axis=2)
    merged = jnp.matmul(A, X).reshape(npat, NF2)
    bft = jnp.concatenate([x, merged], axis=1)
    out = bft @ Wo + bo
    return jax.nn.log_softmax(out, axis=1)


# speedup vs baseline: 2.7523x; 1.3108x over previous
import functools

import jax
import jax.numpy as jnp
from jax import lax
from jax.experimental import pallas as pl
from jax.experimental.pallas import tpu as pltpu
from jax.experimental.pallas import tpu_sc as plsc

NF = 128
NF2 = 256
DK = 64
T = 16

NP = 10240
NTILES = 32
EPT = 5120
EP = EPT * NTILES
CHUNK = 128
NCH = EPT // CHUNK

_HI = lax.Precision.HIGHEST


def _lstm_body(x_ref, web_ref, beb_ref, wih_ref, whh_ref, bl_ref, out_ref):
    B = out_ref.shape[0]
    web = web_ref[...]
    wih = wih_ref[...]
    whh = whh_ref[...]
    beb = beb_ref[...]
    bl = bl_ref[...]
    h = jnp.zeros((B, NF), jnp.float32)
    c = jnp.zeros((B, NF), jnp.float32)
    for t in range(T):
        xt = x_ref[t]
        ht = jnp.maximum(jnp.dot(xt, web, precision=_HI) + beb, 0.0)
        g = (jnp.dot(ht, wih, precision=_HI)
             + jnp.dot(h, whh, precision=_HI) + bl)
        i = jax.nn.sigmoid(g[:, :NF])
        f = jax.nn.sigmoid(g[:, NF:2 * NF])
        gg = jnp.tanh(g[:, 2 * NF:3 * NF])
        o = jax.nn.sigmoid(g[:, 3 * NF:])
        c = f * c + i * gg
        h = o * jnp.tanh(c)
    out_ref[...] = h


def _lstm_last(x, W_eb, b_eb, W_ih, W_hh, b_lstm):
    npat = x.shape[0]
    B = 1000
    xT = jnp.transpose(x, (1, 0, 2))
    grid = (npat // B,)
    return pl.pallas_call(
        _lstm_body,
        grid=grid,
        in_specs=[
            pl.BlockSpec((T, B, NF), lambda g: (0, g, 0)),
            pl.BlockSpec((NF, NF), lambda g: (0, 0)),
            pl.BlockSpec((1, NF), lambda g: (0, 0)),
            pl.BlockSpec((NF, 4 * NF), lambda g: (0, 0)),
            pl.BlockSpec((NF, 4 * NF), lambda g: (0, 0)),
            pl.BlockSpec((1, 4 * NF), lambda g: (0, 0)),
        ],
        out_specs=pl.BlockSpec((B, NF), lambda g: (g, 0)),
        out_shape=jax.ShapeDtypeStruct((npat, NF), jnp.float32),
    )(xT, W_eb, b_eb.reshape(1, NF), W_ih, W_hh, b_lstm.reshape(1, 4 * NF))


def _zmm_body(x_ref, w_ref, a2_ref, z0_ref, z1_ref, z2_ref, z3_ref, ea_ref):
    z = jnp.dot(x_ref[...], w_ref[...], precision=_HI)
    z0_ref[...] = z[:, 0:64]
    z1_ref[...] = z[:, 64:128]
    z2_ref[...] = z[:, 128:192]
    z3_ref[...] = z[:, 192:256]
    ea_ref[...] = jnp.dot(z, a2_ref[...], precision=_HI)


def _gat_dense(x_pad, W, a_l, a_r):
    B = 512
    a2 = jnp.zeros((NF2, 128), jnp.float32)
    a2 = a2.at[:, 0].set(a_l).at[:, 1].set(a_r)
    zq = pl.pallas_call(
        _zmm_body,
        grid=(NP // B,),
        in_specs=[
            pl.BlockSpec((B, NF2), lambda g: (g, 0)),
            pl.BlockSpec((NF2, NF2), lambda g: (0, 0)),
            pl.BlockSpec((NF2, 128), lambda g: (0, 0)),
        ],
        out_specs=[pl.BlockSpec((B, 64), lambda g: (g, 0))] * 4
        + [pl.BlockSpec((B, 128), lambda g: (g, 0))],
        out_shape=[jax.ShapeDtypeStruct((NP, 64), jnp.float32)] * 4
        + [jax.ShapeDtypeStruct((NP, 128), jnp.float32)],
    )(x_pad, W, a2)
    return zq[0], zq[1], zq[2], zq[3], zq[4][:, 0], zq[4][:, 1]


def _edge_body(z0, z1, z2, z3, el_h, er_h, src_h, dst_h,
               oq0, oq1, oq2, oq3, denp,
               src_v, dst_v, el_v, er_v, ex_v, den_v, sidx_v, didx_v,
               sidx2_v, didx2_v, rows_v, rows2_v, zero_v, acc_sh, sem, sem2):
    cid = lax.axis_index("c")
    sid = lax.axis_index("s")
    tid = sid * 2 + cid
    base = tid * EPT
    pltpu.sync_copy(src_h.at[pl.ds(base, EPT)], src_v)
    pltpu.sync_copy(dst_h.at[pl.ds(base, EPT)], dst_v)
    pltpu.sync_copy(el_h, el_v)
    pltpu.sync_copy(er_h, er_v)

    zeros16 = jnp.zeros((16,), jnp.float32)

    def _zden(i, carry):
        den_v[pl.ds(i * 16, 16)] = zeros16
        return carry

    lax.fori_loop(0, NP // 16, _zden, 0)

    def _zrow(r, carry):
        for j in range(4):
            zero_v[r, pl.ds(j * 16, 16)] = zeros16
        return carry

    lax.fori_loop(0, CHUNK, _zrow, 0)

    def _pa(i, carry):
        s = src_v[pl.ds(i * 16, 16)]
        d = dst_v[pl.ds(i * 16, 16)]
        e = plsc.load_gather(el_v, [s]) + plsc.load_gather(er_v, [d])
        e = jnp.where(e > 0, e, 0.2 * e)
        ex = jnp.exp(e)
        ex_v[pl.ds(i * 16, 16)] = ex
        plsc.addupdate_scatter(den_v, [d], ex)
        return carry

    lax.fori_loop(0, EPT // 16, _pa, 0)
    pltpu.sync_copy(den_v, denp.at[tid])

    for h in range(4):
        zh = (z0, z1, z2, z3)[h]
        plsc.subcore_barrier()
        for j in range(5):
            pltpu.sync_copy(zero_v, acc_sh.at[pl.ds((sid * 5 + j) * CHUNK, CHUNK)])
        plsc.subcore_barrier()

        def _stage(c, sidx, didx):
            for j in range(CHUNK // 16):
                sidx[pl.ds(j * 16, 16)] = src_v[pl.ds(c * CHUNK + j * 16, 16)]
                didx[pl.ds(j * 16, 16)] = dst_v[pl.ds(c * CHUNK + j * 16, 16)]

        def _consume(c, rows, didx, s):
            pltpu.make_async_copy(zh.at[pl.ds(0, CHUNK)], rows, s).wait()

            def _row(r, carry2):
                sp = plsc.load_gather(ex_v, [lax.broadcast(c * CHUNK + r, (16,))])
                for j in range(4):
                    rows[r, pl.ds(j * 16, 16)] = rows[r, pl.ds(j * 16, 16)] * sp
                return carry2

            lax.fori_loop(0, CHUNK, _row, 0)
            pltpu.sync_copy(rows, acc_sh.at[didx], add=True)

        _stage(0, sidx_v, didx_v)
        pltpu.async_copy(zh.at[sidx_v], rows_v, sem)

        def _pb2(cp, carry):
            c0 = 2 * cp
            c1 = c0 + 1
            _stage(c1, sidx2_v, didx2_v)
            pltpu.async_copy(zh.at[sidx2_v], rows2_v, sem2)
            _consume(c0, rows_v, didx_v, sem)

            @pl.when(c1 + 1 < NCH)
            def _():
                _stage(c1 + 1, sidx_v, didx_v)
                pltpu.async_copy(zh.at[sidx_v], rows_v, sem)

            _consume(c1, rows2_v, didx2_v, sem2)
            return carry

        lax.fori_loop(0, NCH // 2, _pb2, 0)
        plsc.subcore_barrier()
        oq = (oq0, oq1, oq2, oq3)[h]
        pltpu.sync_copy(
            acc_sh.at[pl.ds(sid * (NP // 16), NP // 16)],
            oq.at[cid, pl.ds(sid * (NP // 16), NP // 16)])
    plsc.subcore_barrier()


@functools.cache
def _edge_sc_kernel():
  return pl.kernel(
    _edge_body,
    out_type=(
        jax.ShapeDtypeStruct((2, NP, 64), jnp.float32),
        jax.ShapeDtypeStruct((2, NP, 64), jnp.float32),
        jax.ShapeDtypeStruct((2, NP, 64), jnp.float32),
        jax.ShapeDtypeStruct((2, NP, 64), jnp.float32),
        jax.ShapeDtypeStruct((NTILES, NP), jnp.float32),
    ),
    mesh=plsc.VectorSubcoreMesh(core_axis_name="c", subcore_axis_name="s"),
    compiler_params=pltpu.CompilerParams(needs_layout_passes=False,
                                         use_tc_tiling_on_sc=False),
    scratch_types=[
        pltpu.VMEM((EPT,), jnp.int32),
        pltpu.VMEM((EPT,), jnp.int32),
        pltpu.VMEM((NP,), jnp.float32),
        pltpu.VMEM((NP,), jnp.float32),
        pltpu.VMEM((EPT,), jnp.float32),
        pltpu.VMEM((NP,), jnp.float32),
        pltpu.VMEM((CHUNK,), jnp.int32),
        pltpu.VMEM((CHUNK,), jnp.int32),
        pltpu.VMEM((CHUNK,), jnp.int32),
        pltpu.VMEM((CHUNK,), jnp.int32),
        pltpu.VMEM((CHUNK, 64), jnp.float32),
        pltpu.VMEM((CHUNK, 64), jnp.float32),
        pltpu.VMEM((CHUNK, 64), jnp.float32),
        pltpu.VMEM_SHARED((NP, 64), jnp.float32),
        pltpu.SemaphoreType.DMA,
        pltpu.SemaphoreType.DMA,
    ],
  )


def _lrelu(x, slope=0.01):
    return jnp.where(x > 0, x, slope * x)


def _gat_pallas(x_pad, src_p, dst_p, W, a_l, a_r, b):
    z0, z1, z2, z3, el, er = _gat_dense(x_pad, W, a_l, a_r)
    oq0, oq1, oq2, oq3, denp = _edge_sc_kernel()(
        z0, z1, z2, z3, el, er, src_p, dst_p)
    den = jnp.sum(denp, axis=0)
    den = jnp.where(den > 0, den, 1.0)
    outs = jnp.concatenate(
        [oq0[0] + oq0[1], oq1[0] + oq1[1], oq2[0] + oq2[1], oq3[0] + oq3[1]],
        axis=1)
    out = outs / den[:, None] + b
    return out


def _pad_edges(src, dst):
    pad = EP - src.shape[0]
    padv = jnp.full((pad,), NP - 1, jnp.int32)
    return (jnp.concatenate([src.astype(jnp.int32), padv]),
            jnp.concatenate([dst.astype(jnp.int32), padv]))


def _tala128(mat, idx_i):
    W = mat.shape[1]
    acc = None
    for q in range(W // 128):
        sub = mat[:, q * 128:(q + 1) * 128]
        loc = jnp.clip(idx_i - q * 128, 0, 127)
        g = jnp.take_along_axis(sub, loc, axis=1)
        sel = (idx_i >= q * 128) & (idx_i < (q + 1) * 128)
        acc = g if acc is None else jnp.where(sel, g, acc)
    return acc


def _knn_body(hb_ref, ht_ref, out_ref, d_ref):
    hb = hb_ref[...]
    ht = ht_ref[...]
    B = hb.shape[0]
    NC = ht.shape[1]
    G = 640
    NJ = NC // G
    BIGF = jnp.float32(3.0e38)
    sqr = jnp.sum(hb * hb, axis=1, keepdims=True)
    sqc = jnp.sum(ht * ht, axis=0, keepdims=True)
    colid = lax.broadcasted_iota(jnp.int32, (1, NC), 1).astype(jnp.float32)
    sqc = sqc + jnp.where(colid >= 10000.0, BIGF, 0.0)
    M = None
    for j in range(NJ):
        dj = ((sqr + sqc[:, j * G:(j + 1) * G])
              - 2.0 * jnp.dot(hb, ht[:, j * G:(j + 1) * G], precision=_HI))
        d_ref[:, j * G:(j + 1) * G] = dj
        M = dj if M is None else jnp.minimum(M, dj)
    iog = lax.broadcasted_iota(jnp.int32, (B, G), 1).astype(jnp.float32)
    gs = []
    for k in range(16):
        m = jnp.min(M, axis=1, keepdims=True)
        g = jnp.min(jnp.where(M == m, iog, BIGF), axis=1, keepdims=True)
        gs.append(g)
        M = jnp.where(iog == g, BIGF, M)
    Gm = jnp.concatenate(gs, axis=1)
    Gi = Gm.astype(jnp.int32)
    cs, cols = [], []
    for j in range(NJ):
        dj = d_ref[:, j * G:(j + 1) * G]
        cs.append(_tala128(dj, Gi))
        cols.append(Gm + jnp.float32(j * G))
    C = jnp.concatenate(cs, axis=1)
    COL = jnp.concatenate(cols, axis=1)
    ioc = lax.broadcasted_iota(jnp.int32, (B, 16 * NJ), 1).astype(jnp.float32)
    outs = []
    for k in range(16):
        m = jnp.min(C, axis=1, keepdims=True)
        p = jnp.min(jnp.where(C == m, ioc, BIGF), axis=1, keepdims=True)
        outs.append(_tala128(COL, p.astype(jnp.int32)))
        C = jnp.where(ioc == p, BIGF, C)
    idxf = jnp.concatenate(outs, axis=1)
    out_ref[...] = jnp.concatenate(
        [idxf.astype(jnp.int32), jnp.zeros((B, 112), jnp.int32)], axis=1)


def _knn_topk(aft):
    npat = aft.shape[0]
    hp = jnp.concatenate(
        [aft, jnp.zeros((NP - npat, NF), jnp.float32)], axis=0)
    ht = jnp.transpose(hp)
    B = 400
    out = pl.pallas_call(
        _knn_body,
        grid=(npat // B,),
        in_specs=[pl.BlockSpec((B, NF), lambda g: (g, 0)),
                  pl.BlockSpec((NF, NP), lambda g: (0, 0))],
        out_specs=pl.BlockSpec((B, 128), lambda g: (g, 0)),
        out_shape=jax.ShapeDtypeStruct((npat, 128), jnp.int32),
        scratch_shapes=[pltpu.VMEM((B, NP), jnp.float32)],
    )(hp, ht)
    return out[:, :16]


def kernel(input_tensor, static_tensor, W_eb, b_eb, W_ih, W_hh, b_lstm, Wg, al,
           ar, bg, Wq, bq, Wk, bk, Wo, bo, ei0, ei1, ei2, ei3, ei4, ei5, w):
    npat = input_tensor.shape[0]

    aft = _lstm_last(input_tensor, W_eb, b_eb, W_ih, W_hh, b_lstm)

    idx = _knn_topk(aft)
    src_dy = jnp.repeat(jnp.arange(npat, dtype=jnp.int32), 16)
    dst_dy = idx.reshape(-1)

    x = jnp.concatenate([aft, static_tensor], axis=1)
    x_pad = jnp.concatenate(
        [x, jnp.zeros((NP - npat, NF2), dtype=x.dtype)], axis=0)

    e_dy = _pad_edges(src_dy, dst_dy)
    e0 = _pad_edges(ei0[0], ei0[1])
    e1 = _pad_edges(ei1[0], ei1[1])
    e2 = _pad_edges(ei2[0], ei2[1])
    e3 = _pad_edges(ei3[0], ei3[1])
    e4 = _pad_edges(ei4[0], ei4[1])
    e5 = _pad_edges(ei5[0], ei5[1])

    aft_dy = _lrelu(_gat_pallas(x_pad, *e_dy, Wg[0], al[0], ar[0], bg[0]))
    g1 = _lrelu(_gat_pallas(x_pad, *e0, Wg[1], al[1], ar[1], bg[1]))
    g2 = _lrelu(_gat_pallas(x_pad, *e1, Wg[2], al[2], ar[2], bg[2]))
    g3 = _lrelu(_gat_pallas(x_pad, *e2, Wg[3], al[3], ar[3], bg[3]))
    g3 = _lrelu(_gat_pallas(g3, *e3, Wg[4], al[4], ar[4], bg[4]))
    g4 = _lrelu(_gat_pallas(x_pad, *e4, Wg[5], al[5], ar[5], bg[5]))
    g4 = _lrelu(_gat_pallas(g4, *e5, Wg[6], al[6], ar[6], bg[6]))

    X = jnp.stack([g1[:npat], g2[:npat], g3[:npat], g4[:npat],
                   aft_dy[:npat]], axis=1)
    hq = x[:, None, :]
    Q = hq @ Wq + bq
    K = X @ Wk + bk
    A = jax.nn.softmax(
        jnp.matmul(Q, jnp.swapaxes(K, -1, -2)) / jnp.sqrt(jnp.float32(DK)),
        axis=2)
    merged = jnp.matmul(A, X).reshape(npat, NF2)
    bft = jnp.concatenate([x, merged], axis=1)
    out = bft @ Wo + bo
    return jax.nn.log_softmax(out, axis=1)
